# baseline (speedup 1.0000x reference)
import functools

import jax
import jax.numpy as jnp
from jax import lax
from jax.experimental import pallas as pl
from jax.experimental.pallas import tpu as pltpu
from jax.experimental.pallas import tpu_sc as plsc

A_DIMS = 100000
BATCH = 16384
ROW = 64
NC, NS, L = 2, 16, 16
NW = NC * NS
BPW = BATCH // NW
NCHUNK = 4
CHUNK = BPW // NCHUNK

_mesh = plsc.VectorSubcoreMesh(core_axis_name="c", subcore_axis_name="s")


@functools.partial(
    pl.kernel,
    out_type=jax.ShapeDtypeStruct((BATCH,), jnp.float32),
    mesh=_mesh,
    compiler_params=pltpu.CompilerParams(
        needs_layout_passes=False, use_tc_tiling_on_sc=False),
    scratch_types=[
        pltpu.VMEM((NCHUNK, CHUNK), jnp.int32),
        pltpu.VMEM((NCHUNK, CHUNK), jnp.int32),
        pltpu.VMEM((NCHUNK, CHUNK), jnp.int32),
        pltpu.VMEM((NCHUNK, CHUNK), jnp.int32),
        pltpu.VMEM((BPW, ROW), jnp.float32),
        pltpu.VMEM((BPW, ROW), jnp.float32),
        pltpu.VMEM((BPW,), jnp.float32),
        pltpu.VMEM((BPW,), jnp.float32),
        pltpu.SemaphoreType.DMA,
        pltpu.SemaphoreType.DMA,
    ],
)
def _sc_kernel(a_hbm, bh0_hbm, bh1_hbm, def_hbm, w_hbm, b_hbm, out_hbm,
               a_v, b0_v, b1_v, idx_v, rows_v, d_v, bg_v, o_v,
               sem_w, sem_b):
    wid = lax.axis_index("s") * NC + lax.axis_index("c")
    base = wid * BPW

    pltpu.sync_copy(a_hbm.at[wid], a_v)
    pltpu.sync_copy(bh0_hbm.at[wid], b0_v)
    pltpu.sync_copy(bh1_hbm.at[wid], b1_v)

    for j in range(NCHUNK):
        for k in range(CHUNK // L):
            s = pl.ds(k * L, L)
            av = a_v[j, s]
            idx_v[j, s] = av * 16 + b0_v[j, s] * 4 + b1_v[j, s]

    copies = []
    for j in range(NCHUNK):
        copies.append(pltpu.async_copy(
            w_hbm.at[idx_v.at[j]], rows_v.at[pl.ds(j * CHUNK, CHUNK)], sem_w))
        copies.append(pltpu.async_copy(
            b_hbm.at[a_v.at[j]], bg_v.at[pl.ds(j * CHUNK, CHUNK)], sem_b))

    pltpu.sync_copy(def_hbm.at[pl.ds(base, BPW)], d_v)
    for c in copies:
        c.wait()

    lane = lax.iota(jnp.int32, L)

    def group(g, carry):
        rbase = g * L
        outv = jnp.zeros((L,), jnp.float32)
        for u in range(L):
            r = rbase + u
            acc = rows_v[r, pl.ds(0, L)] * d_v[r, pl.ds(0, L)]
            for k in range(1, ROW // L):
                acc = acc + rows_v[r, pl.ds(k * L, L)] * d_v[r, pl.ds(k * L, L)]
            outv = jnp.where(lane == u, jnp.sum(acc), outv)
        o_v[pl.ds(rbase, L)] = outv + bg_v[pl.ds(rbase, L)]
        return carry

    lax.fori_loop(0, BPW // L, group, 0)

    pltpu.sync_copy(o_v, out_hbm.at[pl.ds(base, BPW)])


def kernel(a, bh_pos, def_pos, W, b):
    a3 = a.astype(jnp.int32).reshape(NW, NCHUNK, CHUNK)
    bh0 = bh_pos[:, 0].astype(jnp.int32).reshape(NW, NCHUNK, CHUNK)
    bh1 = bh_pos[:, 1].astype(jnp.int32).reshape(NW, NCHUNK, CHUNK)
    d2 = def_pos.astype(jnp.float32).reshape(BATCH, ROW)
    wf = W.reshape(A_DIMS * 16, ROW)
    return _sc_kernel(a3, bh0, bh1, d2, wf, b)


# speedup vs baseline: 17.4328x; 17.4328x over previous
import functools

import jax
import jax.numpy as jnp
from jax import lax
from jax.experimental import pallas as pl
from jax.experimental.pallas import tpu as pltpu
from jax.experimental.pallas import tpu_sc as plsc

A_DIMS = 100000
BATCH = 16384
ROW = 64
RPB = 1024
TCOL = 128
CLAST = (A_DIMS // TCOL) * TCOL
PART = A_DIMS - CLAST
NFULL = A_DIMS // TCOL - 1
NC, NS, L = 2, 16, 16
NW = NC * NS
BPW = BATCH // NW
NCHUNK = 4
CHUNK = BPW // NCHUNK
NBUF = 7
DPW = BPW * ROW

_mesh = plsc.VectorSubcoreMesh(core_axis_name="c", subcore_axis_name="s")


@functools.partial(
    pl.kernel,
    out_type=jax.ShapeDtypeStruct((BATCH,), jnp.float32),
    mesh=_mesh,
    compiler_params=pltpu.CompilerParams(needs_layout_passes=False),
    scratch_types=[
        pltpu.VMEM((BPW + L,), jnp.int32),
        pltpu.VMEM((BPW + L,), jnp.int32),
        pltpu.VMEM((NCHUNK, CHUNK), jnp.int32),
        pltpu.VMEM((NBUF, ROW, TCOL), jnp.float32),
        pltpu.VMEM((RPB * PART,), jnp.float32),
        pltpu.VMEM((DPW,), jnp.float32),
        pltpu.VMEM((BPW,), jnp.float32),
        pltpu.VMEM((BPW,), jnp.float32),
        pltpu.SemaphoreType.DMA((NBUF,)),
        pltpu.SemaphoreType.DMA,
        pltpu.SemaphoreType.DMA,
    ],
)
def _sc_kernel(r0_hbm, a_hbm, def_hbm, w_hbm, wtail_hbm, b_hbm, out_hbm,
               r0_v, a_v, idx_v, blk_v, wt_v, d_v, bg_v, o_v,
               sem_w, sem_b, sem_in):
    wid = lax.axis_index("s") * NC + lax.axis_index("c")
    base = wid * BPW

    for j in range(NCHUNK):
        pltpu.async_copy(
            a_hbm.at[pl.ds(base + j * CHUNK, CHUNK)], idx_v.at[j], sem_in)
    pltpu.sync_copy(r0_hbm.at[pl.ds(base, BPW)], r0_v.at[pl.ds(0, BPW)])
    pltpu.sync_copy(a_hbm.at[pl.ds(base, BPW)], a_v.at[pl.ds(0, BPW)])

    def sread(ref, e):
        return ref[pl.ds(e, L)][0]

    dcp = pltpu.async_copy(def_hbm.at[pl.ds(base * ROW, DPW)], d_v, sem_in)
    wtcp = pltpu.async_copy(wtail_hbm, wt_v, sem_in)
    for j in range(NCHUNK):
        pltpu.make_async_copy(
            a_hbm.at[pl.ds(0, CHUNK)], idx_v.at[j], sem_in).wait()

    bcps = [
        pltpu.async_copy(
            b_hbm.at[idx_v.at[j]], bg_v.at[pl.ds(j * CHUNK, CHUNK)], sem_b)
        for j in range(NCHUNK)
    ]

    def fire(e, buf):
        r0 = pl.multiple_of(sread(r0_v, e), ROW)
        c0 = pl.multiple_of(
            jnp.minimum(sread(a_v, e) // TCOL, NFULL) * TCOL, TCOL)
        pltpu.async_copy(
            w_hbm.at[pl.ds(r0, ROW), pl.ds(c0, TCOL)], blk_v.at[buf],
            sem_w.at[buf])

    def wait_blk(buf):
        pltpu.make_async_copy(
            w_hbm.at[pl.ds(0, ROW), pl.ds(0, TCOL)], blk_v.at[buf],
            sem_w.at[buf]).wait()

    lane = lax.iota(jnp.int32, L)

    def compute(e, buf, outv):
        ae = sread(a_v, e)
        dchunks = [d_v[pl.ds(e * ROW + k * L, L)] for k in range(ROW // L)]

        def from_blk():
            col = jnp.full((L,), ae % TCOL, jnp.int32)
            blk = blk_v.at[buf]
            acc = plsc.load_gather(blk, [lane, col]) * dchunks[0]
            for k in range(1, ROW // L):
                acc = acc + (plsc.load_gather(blk, [k * L + lane, col])
                             * dchunks[k])
            return acc

        def from_tail():
            idx0 = jnp.full((L,), sread(r0_v, e) * PART + (ae - CLAST),
                            jnp.int32)
            fidx = idx0 + lane * PART
            acc = plsc.load_gather(wt_v, [fidx]) * dchunks[0]
            for k in range(1, ROW // L):
                acc = acc + (plsc.load_gather(wt_v, [fidx + k * L * PART])
                             * dchunks[k])
            return acc

        acc = lax.cond(ae < CLAST, from_blk, from_tail)
        return jnp.where(lane == e % L, jnp.sum(acc), outv)

    for e in range(NBUF):
        fire(e, e)
    for c in bcps:
        c.wait()
    dcp.wait()
    wtcp.wait()

    def body(e, outv):
        buf = e % NBUF
        wait_blk(buf)
        outv = compute(e, buf, outv)
        fire(e + NBUF, buf)

        @pl.when(e % L == L - 1)
        def _():
            g = e - (L - 1)
            o_v[pl.ds(g, L)] = outv + bg_v[pl.ds(g, L)]

        return outv

    outv = lax.fori_loop(0, BPW - NBUF, body, jnp.zeros((L,), jnp.float32))

    def tail_loop(e, outv):
        buf = e % NBUF
        wait_blk(buf)
        outv = compute(e, buf, outv)

        @pl.when(e % L == L - 1)
        def _():
            g = e - (L - 1)
            o_v[pl.ds(g, L)] = outv + bg_v[pl.ds(g, L)]

        return outv

    lax.fori_loop(BPW - NBUF, BPW, tail_loop, outv)

    pltpu.sync_copy(o_v, out_hbm.at[pl.ds(base, BPW)])


def kernel(a, bh_pos, def_pos, W, b):
    a32 = a.astype(jnp.int32)
    r0 = (bh_pos[:, 0].astype(jnp.int32) * 4 + bh_pos[:, 1].astype(jnp.int32)) * ROW
    deff = def_pos.astype(jnp.float32).reshape(BATCH * ROW)
    wp = W.transpose(1, 2, 3, 4, 0).reshape(RPB, A_DIMS)
    wtail = wp[:, CLAST:].reshape(RPB * PART)
    return _sc_kernel(r0, a32, deff, wp, wtail, b)
